# baseline (device time: 15428 ns/iter reference)
import jax
import jax.numpy as jnp
from jax import lax
from jax.experimental import pallas as pl
from jax.experimental.pallas import tpu as pltpu


def kernel(ids, E):
    v_per, d = E.shape
    t = ids.shape[0]
    q = t // 4
    h = q // 2

    my_y = lax.axis_index("y")
    local = (ids - my_y * v_per).astype(jnp.int32)
    raw = E.at[jnp.bitwise_and(local, v_per - 1), :].get(
        mode="promise_in_bounds"
    )

    def body(raw_ref, lv_ref, out_ref, gbuf, fresh_ref, mbuf,
             gsem, osem, y_ssem, y_rsem, f_ssem, f_rsem):
        mx = lax.axis_index("x")
        my = lax.axis_index("y")
        mz = lax.axis_index("z")
        y_nbr = (mx, 1 - my, mz)
        x_nbr = (1 - mx, my, mz)
        z_nbr = (mx, my, 1 - mz)
        aa = 2 * mx + mz
        bb = 3 - aa

        barrier = pltpu.get_barrier_semaphore()
        for nbr in (y_nbr, x_nbr, z_nbr):
            pl.semaphore_signal(
                barrier, inc=1, device_id=nbr,
                device_id_type=pl.DeviceIdType.MESH,
            )

        chunks = ((0, h), (h, h), (q, q))
        g_off = (aa * q, aa * q + h, bb * q)
        pulls = []
        for i, (off, ln) in enumerate(chunks):
            cp = pltpu.make_async_copy(
                raw_ref.at[pl.ds(g_off[i], ln), :],
                gbuf.at[pl.ds(off, ln), :],
                gsem.at[i],
            )
            cp.start()
            pulls.append(cp)

        pl.semaphore_wait(barrier, 3)

        y_sends = []
        for i, (off, ln) in enumerate(chunks):
            pulls[i].wait()
            snd = pltpu.make_async_remote_copy(
                src_ref=gbuf.at[pl.ds(off, ln), :],
                dst_ref=fresh_ref.at[pl.ds(off, ln), :],
                send_sem=y_ssem.at[i], recv_sem=y_rsem.at[i],
                device_id=y_nbr, device_id_type=pl.DeviceIdType.MESH,
            )
            snd.start()
            y_sends.append(snd)

        out_dmas = []

        def merge(off, ln, out_off, i):
            lv = lv_ref[pl.ds(out_off, ln), :]
            valid = (lv >= 0) & (lv < v_per)
            mbuf[pl.ds(off, ln), :] = jnp.where(
                valid,
                gbuf[pl.ds(off, ln), :],
                fresh_ref[pl.ds(off, ln), :],
            )
            dma = pltpu.make_async_copy(
                mbuf.at[pl.ds(off, ln), :],
                out_ref.at[pl.ds(out_off, ln), :],
                osem.at[i],
            )
            dma.start()
            out_dmas.append(dma)

        fwds = []
        for c in range(2):
            y_sends[c].wait_recv()
            off = c * h
            merge(off, h, aa * q + off, c)
            for j, nbr in enumerate((x_nbr, z_nbr)):
                k = 2 * c + j
                fwd = pltpu.make_async_remote_copy(
                    src_ref=mbuf.at[pl.ds(off, h), :],
                    dst_ref=out_ref.at[pl.ds(aa * q + off, h), :],
                    send_sem=f_ssem.at[k], recv_sem=f_rsem.at[k],
                    device_id=nbr, device_id_type=pl.DeviceIdType.MESH,
                )
                fwd.start()
                fwds.append(fwd)

        y_sends[2].wait_recv()
        merge(q, q, bb * q, 2)

        for fwd in fwds:
            fwd.wait_recv()
        for dma in out_dmas:
            dma.wait()
        for snd in y_sends:
            snd.wait_send()
        for fwd in fwds:
            fwd.wait_send()

    return pl.pallas_call(
        body,
        out_shape=jax.ShapeDtypeStruct((t, d), jnp.float32),
        in_specs=[
            pl.BlockSpec(memory_space=pl.ANY),
            pl.BlockSpec(memory_space=pltpu.VMEM),
        ],
        out_specs=pl.BlockSpec(memory_space=pltpu.MemorySpace.HBM),
        scratch_shapes=[
            pltpu.VMEM((2 * q, d), jnp.float32),
            pltpu.VMEM((2 * q, d), jnp.float32),
            pltpu.VMEM((2 * q, d), jnp.float32),
            pltpu.SemaphoreType.DMA((3,)),
            pltpu.SemaphoreType.DMA((3,)),
            pltpu.SemaphoreType.DMA((3,)),
            pltpu.SemaphoreType.DMA((3,)),
            pltpu.SemaphoreType.DMA((4,)),
            pltpu.SemaphoreType.DMA((4,)),
        ],
        compiler_params=pltpu.CompilerParams(collective_id=0),
    )(raw, local[:, None])


# device time: 14681 ns/iter; 1.0509x vs baseline; 1.0509x over previous
import jax
import jax.numpy as jnp
from jax import lax
from jax.experimental import pallas as pl
from jax.experimental.pallas import tpu as pltpu


def kernel(ids, E):
    v_per, d = E.shape
    t = ids.shape[0]
    q = t // 4
    ha = q // 4
    hb = q // 2

    my_y = lax.axis_index("y")
    local = (ids - my_y * v_per).astype(jnp.int32)
    raw = E[jnp.bitwise_and(local, v_per - 1), :]

    chunks = tuple((i * ha, ha) for i in range(4)) + ((q, hb), (q + hb, hb))

    def body(raw_ref, lv_ref, out_ref, gbuf, fresh_ref,
             gsem, y_ssem, y_rsem, f_ssem, f_rsem):
        mx = lax.axis_index("x")
        my = lax.axis_index("y")
        mz = lax.axis_index("z")
        y_nbr = (mx, 1 - my, mz)
        x_nbr = (1 - mx, my, mz)
        z_nbr = (mx, my, 1 - mz)
        aa = 2 * mx + mz
        bb = 3 - aa

        barrier = pltpu.get_barrier_semaphore()
        for nbr in (y_nbr, x_nbr, z_nbr):
            pl.semaphore_signal(
                barrier, inc=1, device_id=nbr,
                device_id_type=pl.DeviceIdType.MESH,
            )

        pulls = []
        for i, src in enumerate((aa * q, bb * q)):
            cp = pltpu.make_async_copy(
                raw_ref.at[pl.ds(src, q), :],
                gbuf.at[pl.ds(i * q, q), :],
                gsem.at[i],
            )
            cp.start()
            pulls.append(cp)

        pl.semaphore_wait(barrier, 3)

        y_sends = []
        for i, (off, ln) in enumerate(chunks):
            if i == 0:
                pulls[0].wait()
            if i == 4:
                pulls[1].wait()
            snd = pltpu.make_async_remote_copy(
                src_ref=gbuf.at[pl.ds(off, ln), :],
                dst_ref=fresh_ref.at[pl.ds(off, ln), :],
                send_sem=y_ssem.at[i], recv_sem=y_rsem.at[i],
                device_id=y_nbr, device_id_type=pl.DeviceIdType.MESH,
            )
            snd.start()
            y_sends.append(snd)

        def merge(off, ln, out_off):
            lv = lv_ref[pl.ds(out_off, ln), :]
            valid = (lv >= 0) & (lv < v_per)
            out_ref[pl.ds(out_off, ln), :] = jnp.where(
                valid,
                gbuf[pl.ds(off, ln), :],
                fresh_ref[pl.ds(off, ln), :],
            )

        fwds = []
        for c in range(4):
            y_sends[c].wait_recv()
            off = c * ha
            merge(off, ha, aa * q + off)
            for j, nbr in enumerate((x_nbr, z_nbr)):
                k = 2 * c + j
                fwd = pltpu.make_async_remote_copy(
                    src_ref=out_ref.at[pl.ds(aa * q + off, ha), :],
                    dst_ref=out_ref.at[pl.ds(aa * q + off, ha), :],
                    send_sem=f_ssem.at[k], recv_sem=f_rsem.at[k],
                    device_id=nbr, device_id_type=pl.DeviceIdType.MESH,
                )
                fwd.start()
                fwds.append(fwd)

        for c in range(2):
            y_sends[4 + c].wait_recv()
            merge(q + c * hb, hb, bb * q + c * hb)

        for fwd in fwds:
            fwd.wait_recv()
        for snd in y_sends:
            snd.wait_send()
        for fwd in fwds:
            fwd.wait_send()

    return pl.pallas_call(
        body,
        out_shape=jax.ShapeDtypeStruct((t, d), jnp.float32),
        in_specs=[
            pl.BlockSpec(memory_space=pl.ANY),
            pl.BlockSpec(memory_space=pltpu.VMEM),
        ],
        out_specs=pl.BlockSpec(memory_space=pltpu.VMEM),
        scratch_shapes=[
            pltpu.VMEM((2 * q, d), jnp.float32),
            pltpu.VMEM((2 * q, d), jnp.float32),
            pltpu.SemaphoreType.DMA((2,)),
            pltpu.SemaphoreType.DMA((6,)),
            pltpu.SemaphoreType.DMA((6,)),
            pltpu.SemaphoreType.DMA((8,)),
            pltpu.SemaphoreType.DMA((8,)),
        ],
        compiler_params=pltpu.CompilerParams(collective_id=0),
    )(raw, local[:, None])


# device time: 14638 ns/iter; 1.0540x vs baseline; 1.0029x over previous
import jax
import jax.numpy as jnp
from jax import lax
from jax.experimental import pallas as pl
from jax.experimental.pallas import tpu as pltpu


def kernel(ids, E):
    v_per, d = E.shape
    t = ids.shape[0]
    q = t // 4
    ha = q // 4
    hb = q // 2

    my_y = lax.axis_index("y")
    local = (ids - my_y * v_per).astype(jnp.int32)
    raw = E[jnp.bitwise_and(local, v_per - 1), :]

    def body(raw_ref, lv_ref, out_ref, fresh_ref,
             y_ssem, y_rsem, f_ssem, f_rsem):
        mx = lax.axis_index("x")
        my = lax.axis_index("y")
        mz = lax.axis_index("z")
        y_nbr = (mx, 1 - my, mz)
        x_nbr = (1 - mx, my, mz)
        z_nbr = (mx, my, 1 - mz)
        aa = 2 * mx + mz
        bb = 3 - aa

        chunks = tuple((aa * q + i * ha, ha) for i in range(4)) + tuple(
            (bb * q + c * hb, hb) for c in range(2)
        )

        barrier = pltpu.get_barrier_semaphore()
        for nbr in (y_nbr, x_nbr, z_nbr):
            pl.semaphore_signal(
                barrier, inc=1, device_id=nbr,
                device_id_type=pl.DeviceIdType.MESH,
            )
        pl.semaphore_wait(barrier, 3)

        y_sends = []
        for i, (off, ln) in enumerate(chunks):
            snd = pltpu.make_async_remote_copy(
                src_ref=raw_ref.at[pl.ds(off, ln), :],
                dst_ref=fresh_ref.at[pl.ds(off, ln), :],
                send_sem=y_ssem.at[i], recv_sem=y_rsem.at[i],
                device_id=y_nbr, device_id_type=pl.DeviceIdType.MESH,
            )
            snd.start()
            y_sends.append(snd)

        def merge(off, ln):
            lv = lv_ref[pl.ds(off, ln), :]
            valid = (lv >= 0) & (lv < v_per)
            out_ref[pl.ds(off, ln), :] = jnp.where(
                valid,
                raw_ref[pl.ds(off, ln), :],
                fresh_ref[pl.ds(off, ln), :],
            )

        fwds = []
        for c in range(4):
            y_sends[c].wait_recv()
            off, ln = chunks[c]
            merge(off, ln)
            for j, nbr in enumerate((x_nbr, z_nbr)):
                k = 2 * c + j
                fwd = pltpu.make_async_remote_copy(
                    src_ref=out_ref.at[pl.ds(off, ln), :],
                    dst_ref=out_ref.at[pl.ds(off, ln), :],
                    send_sem=f_ssem.at[k], recv_sem=f_rsem.at[k],
                    device_id=nbr, device_id_type=pl.DeviceIdType.MESH,
                )
                fwd.start()
                fwds.append(fwd)

        for c in range(2):
            y_sends[4 + c].wait_recv()
            merge(*chunks[4 + c])

        for fwd in fwds:
            fwd.wait_recv()
        for snd in y_sends:
            snd.wait_send()
        for fwd in fwds:
            fwd.wait_send()

    return pl.pallas_call(
        body,
        out_shape=jax.ShapeDtypeStruct((t, d), jnp.float32),
        in_specs=[
            pl.BlockSpec(memory_space=pltpu.VMEM),
            pl.BlockSpec(memory_space=pltpu.VMEM),
        ],
        out_specs=pl.BlockSpec(memory_space=pltpu.VMEM),
        scratch_shapes=[
            pltpu.VMEM((t, d), jnp.float32),
            pltpu.SemaphoreType.DMA((6,)),
            pltpu.SemaphoreType.DMA((6,)),
            pltpu.SemaphoreType.DMA((8,)),
            pltpu.SemaphoreType.DMA((8,)),
        ],
        compiler_params=pltpu.CompilerParams(collective_id=0),
    )(raw, local[:, None])
